# baseline (speedup 1.0000x reference)
import functools

import jax
import jax.numpy as jnp
from jax import lax
from jax.experimental import pallas as pl
from jax.experimental.pallas import tpu as pltpu
from jax.experimental.pallas import tpu_sc as plsc

B = 16384
D = 16
H = 50
NI = 200
NIP = 208
NC = 2
NS = 16
NW = NC * NS
RPW = B // NW
HCH = 64
NHC = RPW // HCH
ICH = 16
NIC = RPW // ICH
GPR = NIP // 16

_i32 = jnp.int32
_f32 = jnp.float32


def _lanes():
    return lax.iota(_i32, 16)


def _splat(x):
    return jnp.full((16,), x, _i32)


def _sc_gather_body(discf_h, histf_h, item_h, user_h, prov_h, city_h,
                    user_o, prov_o, city_o, histf_o,
                    discf_v, uidx_v, pidx_v, cidx_v, srows_v, hidx_v, hrows_v,
                    zbuf_v, hbuf_v, t0_v, sem):
    wid = lax.axis_index("s") * NC + lax.axis_index("c")
    base = pl.multiple_of(wid * RPW, RPW)
    lanes = _lanes()

    pltpu.sync_copy(item_h.at[0], t0_v)

    pltpu.sync_copy(discf_h.at[pl.ds(base * 5, RPW * 5)], discf_v)

    def colbody(g, _):
        flat16 = g * 80 + lanes * 5
        for buf, c in ((uidx_v, 0), (pidx_v, 3), (cidx_v, 4)):
            v = plsc.load_gather(discf_v, [flat16 + c])
            buf[pl.ds(pl.multiple_of(g * 16, 16), 16)] = v
        return 0

    lax.fori_loop(0, RPW // 16, colbody, 0)

    for idxbuf, tab, out in ((uidx_v, user_h, user_o), (pidx_v, prov_h, prov_o),
                             (cidx_v, city_h, city_o)):
        cps = [
            pltpu.async_copy(tab.at[idxbuf.at[pl.ds(j * 128, 128)]],
                             srows_v.at[pl.ds(j * 128, 128)], sem)
            for j in range(RPW // 128)
        ]
        for cp in cps:
            cp.wait()
        pltpu.sync_copy(srows_v, out.at[pl.ds(base, RPW)])

    def hist_chunk(cc, _):
        foff = pl.multiple_of(base * H + cc * (HCH * H), 8)
        pltpu.sync_copy(histf_h.at[pl.ds(foff, HCH * H)], hidx_v)
        cps = [
            pltpu.async_copy(item_h.at[hidx_v.at[pl.ds(j * 128, 128)]],
                             hrows_v.at[pl.ds(j * 128, 128)], sem)
            for j in range(HCH * H // 128)
        ]
        for cp in cps:
            cp.wait()

        def zbody(rr, _):
            rbase = pl.multiple_of(rr * 16, 16)
            roff = (rbase + lanes) * H
            zacc = jnp.zeros((16,), _f32)
            for j in range(H):
                iv = plsc.load_gather(hidx_v, [roff + j])
                zacc = zacc + jnp.where(iv == 0, 1.0, 0.0)
            zbuf_v[pl.ds(rbase, 16)] = zacc
            return 0

        lax.fori_loop(0, HCH // 16, zbody, 0)

        def rowbody(r, _):
            off = r * H
            racc = jnp.zeros((16,), _f32)
            for j in range(H):
                racc = racc + hrows_v[off + j, :]
            zr = plsc.load_gather(zbuf_v, [_splat(r)])
            hbuf_v[pl.ds(pl.multiple_of(r * D, D), D)] = \
                (racc - zr * t0_v[:]) * (1.0 / H)
            return 0

        lax.fori_loop(0, HCH, rowbody, 0)
        pltpu.sync_copy(
            hbuf_v, histf_o.at[pl.ds((base + cc * HCH) * D, HCH * D)])
        return 0

    lax.fori_loop(0, NHC, hist_chunk, 0)


@functools.cache
def _sc_gather():
  return pl.kernel(
    _sc_gather_body,
    out_type=[jax.ShapeDtypeStruct((B, D), _f32)] * 3
    + [jax.ShapeDtypeStruct((B * D,), _f32)],
    mesh=plsc.VectorSubcoreMesh(core_axis_name="c", subcore_axis_name="s",
                                num_cores=NC, num_subcores=NS),
    compiler_params=pltpu.CompilerParams(needs_layout_passes=False, use_tc_tiling_on_sc=False),
    scratch_types=[
        pltpu.VMEM((RPW * 5,), _i32),
        pltpu.VMEM((RPW,), _i32),
        pltpu.VMEM((RPW,), _i32),
        pltpu.VMEM((RPW,), _i32),
        pltpu.VMEM((RPW, D), _f32),
        pltpu.VMEM((HCH * H,), _i32),
        pltpu.VMEM((HCH * H, D), _f32),
        pltpu.VMEM((HCH,), _f32),
        pltpu.VMEM((HCH * D,), _f32),
        pltpu.VMEM((16,), _f32),
        pltpu.SemaphoreType.DMA,
    ],
  )


def _rsqrt(t):
    bits = plsc.bitcast(t, _i32)
    y = plsc.bitcast(jnp.full((16,), 0x5F3759DF, _i32)
                     - lax.shift_right_logical(bits, 1), _f32)
    ht = t * 0.5
    for _ in range(2):
        y = y * (1.5 - ht * y * y)
    return y


_NPAD = ICH * NI
_NG = _NPAD // 128
_GOFFS = [g * 16 for g in range(NI // 16)] + [NI - 16]


def _sc_items_body(itemsf_h, item_h, uf_h,
                   scoresf_o,
                   u_v, iidx_a, iidx_b, irows_a, irows_b,
                   tbuf0, tbuf1, tbuf2, tbuf3,
                   sbuf_a, sbuf_b, semi_a, semi_b, semg_a, semg_b,
                   semo_a, semo_b):
    tbufs = (tbuf0, tbuf1, tbuf2, tbuf3)
    wid = lax.axis_index("s") * NC + lax.axis_index("c")
    base = pl.multiple_of(wid * RPW, RPW)
    lanes = _lanes()

    pltpu.sync_copy(uf_h.at[pl.ds(base * D, RPW * D)], u_v)

    def idx_src(chunk):
        off = pl.multiple_of(base * NI + chunk * _NPAD, 8)
        return itemsf_h.at[pl.ds(off, _NPAD)]

    def fire_gathers(iidx, irows, sem):
        for j in range(_NG):
            pltpu.async_copy(item_h.at[iidx.at[pl.ds(j * 128, 128)]],
                             irows.at[pl.ds(j * 128, 128)], sem)

    def drain_gathers(iidx, irows, sem):
        for j in range(_NG):
            pltpu.make_async_copy(
                item_h.at[iidx.at[pl.ds(j * 128, 128)]],
                irows.at[pl.ds(j * 128, 128)], sem).wait()

    def out_dst(chunk):
        return scoresf_o.at[pl.ds((base + chunk * ICH) * NI, _NPAD)]

    def compute(chunk, iidx, irows, sbuf):
        def rowbody(r, _):
            rbase = chunk * ICH + r
            ub = [plsc.load_gather(u_v, [_splat(rbase * D + d)])
                  for d in range(D)]
            na2a = ub[0] * ub[0]
            na2b = ub[1] * ub[1]
            for d in range(2, D, 2):
                na2a = na2a + ub[d] * ub[d]
                na2b = na2b + ub[d + 1] * ub[d + 1]
            na2 = na2a + na2b
            lgs = []
            for gi, goff in enumerate(_GOFFS):
                off = pl.multiple_of(r * NI, 8) + goff
                idxv = iidx[pl.ds(off, 16)]
                tbuf = tbufs[gi % 4]
                for k in range(16):
                    rv = irows[off + k, :]
                    plsc.store_scatter(tbuf, [lanes * 16 + k], rv)
                c0 = tbuf[pl.ds(0, 16)]
                c1 = tbuf[pl.ds(16, 16)]
                dot0 = c0 * ub[0]
                dot1 = c1 * ub[1]
                nb0 = c0 * c0
                nb1 = c1 * c1
                for d in range(2, D, 2):
                    c0 = tbuf[pl.ds(d * 16, 16)]
                    c1 = tbuf[pl.ds((d + 1) * 16, 16)]
                    dot0 = dot0 + c0 * ub[d]
                    dot1 = dot1 + c1 * ub[d + 1]
                    nb0 = nb0 + c0 * c0
                    nb1 = nb1 + c1 * c1
                dotacc = dot0 + dot1
                nbacc = nb0 + nb1
                rs = _rsqrt(jnp.maximum(na2 * nbacc, 1e-16))
                lgs.append(jnp.where(idxv != 0, dotacc * rs, 0.0))
            m = lgs[0]
            for lg in lgs[1:]:
                m = jnp.maximum(m, lg)
            mv = jnp.full((16,), jnp.max(m), _f32)
            es = [jnp.exp(lg - mv) for lg in lgs]
            sv = es[0]
            for e in es[1:-1]:
                sv = sv + e
            sv = sv + jnp.where(lanes < 8, 0.0, es[-1])
            rcp = jnp.full((16,), 1.0, _f32) / jnp.full((16,), jnp.sum(sv), _f32)
            for i, goff in enumerate(_GOFFS):
                sbuf[pl.ds(pl.multiple_of(r * NI, 8) + goff, 16)] = es[i] * rcp
            return 0

        lax.fori_loop(0, ICH, rowbody, 0)

    pltpu.async_copy(idx_src(0), iidx_a, semi_a)
    pltpu.async_copy(idx_src(1), iidx_b, semi_b)
    pltpu.make_async_copy(idx_src(0), iidx_a, semi_a).wait()
    fire_gathers(iidx_a, irows_a, semg_a)

    def pipe(cc2, _):
        e = cc2 * 2

        pltpu.make_async_copy(idx_src(1), iidx_b, semi_b).wait()
        fire_gathers(iidx_b, irows_b, semg_b)
        drain_gathers(iidx_a, irows_a, semg_a)

        @pl.when(cc2 < NIC // 2 - 1)
        def _():
            pltpu.async_copy(idx_src(e + 2), iidx_a, semi_a)

        @pl.when(cc2 >= 1)
        def _():
            pltpu.make_async_copy(sbuf_a, out_dst(0), semo_a).wait()
        compute(e, iidx_a, irows_a, sbuf_a)
        pltpu.async_copy(sbuf_a, out_dst(e), semo_a)

        @pl.when(cc2 < NIC // 2 - 1)
        def _():
            pltpu.make_async_copy(idx_src(e + 2), iidx_a, semi_a).wait()
            fire_gathers(iidx_a, irows_a, semg_a)
            pltpu.async_copy(idx_src(e + 3), iidx_b, semi_b)
        drain_gathers(iidx_b, irows_b, semg_b)

        @pl.when(cc2 >= 1)
        def _():
            pltpu.make_async_copy(sbuf_b, out_dst(0), semo_b).wait()
        compute(e + 1, iidx_b, irows_b, sbuf_b)
        pltpu.async_copy(sbuf_b, out_dst(e + 1), semo_b)
        return 0

    lax.fori_loop(0, NIC // 2, pipe, 0)
    pltpu.make_async_copy(sbuf_a, out_dst(0), semo_a).wait()
    pltpu.make_async_copy(sbuf_b, out_dst(0), semo_b).wait()


@functools.cache
def _sc_items():
  return pl.kernel(
    _sc_items_body,
    out_type=jax.ShapeDtypeStruct((B * NI,), _f32),
    mesh=plsc.VectorSubcoreMesh(core_axis_name="c", subcore_axis_name="s",
                                num_cores=NC, num_subcores=NS),
    compiler_params=pltpu.CompilerParams(needs_layout_passes=False, use_tc_tiling_on_sc=False),
    scratch_types=[
        pltpu.VMEM((RPW * D,), _f32),
        pltpu.VMEM((_NPAD,), _i32),
        pltpu.VMEM((_NPAD,), _i32),
        pltpu.VMEM((_NPAD, D), _f32),
        pltpu.VMEM((_NPAD, D), _f32),
        pltpu.VMEM((256,), _f32),
        pltpu.VMEM((256,), _f32),
        pltpu.VMEM((256,), _f32),
        pltpu.VMEM((256,), _f32),
        pltpu.VMEM((_NPAD,), _f32),
        pltpu.VMEM((_NPAD,), _f32),
        pltpu.SemaphoreType.DMA,
        pltpu.SemaphoreType.DMA,
        pltpu.SemaphoreType.DMA,
        pltpu.SemaphoreType.DMA,
        pltpu.SemaphoreType.DMA,
        pltpu.SemaphoreType.DMA,
    ],
  )


_MBLK = 2048


def _mlp_body(user_r, prov_r, city_r, hist_r, cont_r,
              w1_r, b1_r, w2_r, b2_r, w3_r, b3_r, out_r):
    w1 = w1_r[...]
    h = (jnp.dot(user_r[...], w1[0:16, :], preferred_element_type=_f32)
         + jnp.dot(prov_r[...], w1[16:32, :], preferred_element_type=_f32)
         + jnp.dot(city_r[...], w1[32:48, :], preferred_element_type=_f32)
         + jnp.dot(hist_r[...], w1[48:64, :], preferred_element_type=_f32)
         + jnp.dot(cont_r[...], w1[64:92, :], preferred_element_type=_f32)
         + b1_r[...])
    h = jnp.dot(h, w2_r[...], preferred_element_type=_f32) + b2_r[...]
    out_r[...] = jnp.dot(h, w3_r[...], preferred_element_type=_f32) + b3_r[...]


def _row_spec(cols):
    return pl.BlockSpec((_MBLK, cols), lambda i: (i, 0))


def _full_spec(shape):
    return pl.BlockSpec(shape, lambda i: tuple(0 for _ in shape))


_tc_mlp = pl.pallas_call(
    _mlp_body,
    grid=(B // _MBLK,),
    in_specs=[
        _row_spec(D), _row_spec(D), _row_spec(D), _row_spec(D), _row_spec(28),
        _full_spec((92, 128)), _full_spec((1, 128)),
        _full_spec((128, 32)), _full_spec((1, 32)),
        _full_spec((32, 16)), _full_spec((1, 16)),
    ],
    out_specs=_row_spec(D),
    out_shape=jax.ShapeDtypeStruct((B, D), _f32),
)


def kernel(disc, cont, history, items, item_table, user_table, city_table,
           prov_table, dev_table, os_table, W1, b1, W2, b2, W3, b3):
    user_e, prov_e, city_e, histf_e = _sc_gather()(
        disc.reshape(-1), history.reshape(-1), item_table, user_table,
        prov_table, city_table)
    u16 = _tc_mlp(user_e, prov_e, city_e, histf_e.reshape(B, D), cont,
                  W1, b1.reshape(1, -1), W2, b2.reshape(1, -1),
                  W3, b3.reshape(1, -1))
    scoresf = _sc_items()(items.reshape(-1), item_table, u16.reshape(-1))
    return scoresf.reshape(B, NI)


# speedup vs baseline: 1.0252x; 1.0252x over previous
import functools

import jax
import jax.numpy as jnp
from jax import lax
from jax.experimental import pallas as pl
from jax.experimental.pallas import tpu as pltpu
from jax.experimental.pallas import tpu_sc as plsc

B = 16384
D = 16
H = 50
NI = 200
NIP = 208
NC = 2
NS = 16
NW = NC * NS
RPW = B // NW
HCH = 64
NHC = RPW // HCH
ICH = 16
NIC = RPW // ICH
GPR = NIP // 16

_i32 = jnp.int32
_f32 = jnp.float32


def _lanes():
    return lax.iota(_i32, 16)


def _splat(x):
    return jnp.full((16,), x, _i32)


def _sc_gather_body(discf_h, histf_h, item_h, user_h, prov_h, city_h,
                    user_o, prov_o, city_o, histf_o,
                    discf_v, uidx_v, pidx_v, cidx_v, srows_v, hidx_v, hrows_v,
                    hbuf_v, sem):
    wid = lax.axis_index("s") * NC + lax.axis_index("c")
    base = pl.multiple_of(wid * RPW, RPW)
    lanes = _lanes()

    pltpu.sync_copy(discf_h.at[pl.ds(base * 5, RPW * 5)], discf_v)

    def colbody(g, _):
        flat16 = g * 80 + lanes * 5
        for buf, c in ((uidx_v, 0), (pidx_v, 3), (cidx_v, 4)):
            v = plsc.load_gather(discf_v, [flat16 + c])
            buf[pl.ds(pl.multiple_of(g * 16, 16), 16)] = v
        return 0

    lax.fori_loop(0, RPW // 16, colbody, 0)

    for idxbuf, tab, out in ((uidx_v, user_h, user_o), (pidx_v, prov_h, prov_o),
                             (cidx_v, city_h, city_o)):
        cps = [
            pltpu.async_copy(tab.at[idxbuf.at[pl.ds(j * 128, 128)]],
                             srows_v.at[pl.ds(j * 128, 128)], sem)
            for j in range(RPW // 128)
        ]
        for cp in cps:
            cp.wait()
        pltpu.sync_copy(srows_v, out.at[pl.ds(base, RPW)])

    def hist_chunk(cc, _):
        foff = pl.multiple_of(base * H + cc * (HCH * H), 8)
        pltpu.sync_copy(histf_h.at[pl.ds(foff, HCH * H)], hidx_v)
        cps = [
            pltpu.async_copy(item_h.at[hidx_v.at[pl.ds(j * 128, 128)]],
                             hrows_v.at[pl.ds(j * 128, 128)], sem)
            for j in range(HCH * H // 128)
        ]
        for cp in cps:
            cp.wait()

        def rowbody(r, _):
            off = r * H
            racc0 = hrows_v[off, :]
            racc1 = hrows_v[off + 1, :]
            for j in range(2, H, 2):
                racc0 = racc0 + hrows_v[off + j, :]
                racc1 = racc1 + hrows_v[off + j + 1, :]
            hbuf_v[pl.ds(pl.multiple_of(r * D, D), D)] = \
                (racc0 + racc1) * (1.0 / H)
            return 0

        lax.fori_loop(0, HCH, rowbody, 0)
        pltpu.sync_copy(
            hbuf_v, histf_o.at[pl.ds((base + cc * HCH) * D, HCH * D)])
        return 0

    lax.fori_loop(0, NHC, hist_chunk, 0)


@functools.cache
def _sc_gather():
  return pl.kernel(
    _sc_gather_body,
    out_type=[jax.ShapeDtypeStruct((B, D), _f32)] * 3
    + [jax.ShapeDtypeStruct((B * D,), _f32)],
    mesh=plsc.VectorSubcoreMesh(core_axis_name="c", subcore_axis_name="s",
                                num_cores=NC, num_subcores=NS),
    compiler_params=pltpu.CompilerParams(needs_layout_passes=False, use_tc_tiling_on_sc=False),
    scratch_types=[
        pltpu.VMEM((RPW * 5,), _i32),
        pltpu.VMEM((RPW,), _i32),
        pltpu.VMEM((RPW,), _i32),
        pltpu.VMEM((RPW,), _i32),
        pltpu.VMEM((RPW, D), _f32),
        pltpu.VMEM((HCH * H,), _i32),
        pltpu.VMEM((HCH * H, D), _f32),
        pltpu.VMEM((HCH * D,), _f32),
        pltpu.SemaphoreType.DMA,
    ],
  )


def _rsqrt(t):
    bits = plsc.bitcast(t, _i32)
    y = plsc.bitcast(jnp.full((16,), 0x5F3759DF, _i32)
                     - lax.shift_right_logical(bits, 1), _f32)
    ht = t * 0.5
    for _ in range(2):
        y = y * (1.5 - ht * y * y)
    return y


_NPAD = ICH * NI
_NG = _NPAD // 128
_GOFFS = [g * 16 for g in range(NI // 16)] + [NI - 16]


def _sc_items_body(itemsf_h, item_h, uf_h,
                   scoresf_o,
                   u_v, iidx_a, iidx_b, irows_a, irows_b,
                   tbuf0, tbuf1, tbuf2, tbuf3,
                   sbuf_a, sbuf_b, semi_a, semi_b, semg_a, semg_b,
                   semo_a, semo_b):
    tbufs = (tbuf0, tbuf1, tbuf2, tbuf3)
    wid = lax.axis_index("s") * NC + lax.axis_index("c")
    base = pl.multiple_of(wid * RPW, RPW)
    lanes = _lanes()

    pltpu.sync_copy(uf_h.at[pl.ds(base * D, RPW * D)], u_v)

    def idx_src(chunk):
        off = pl.multiple_of(base * NI + chunk * _NPAD, 8)
        return itemsf_h.at[pl.ds(off, _NPAD)]

    def fire_gathers(iidx, irows, sem):
        for j in range(_NG):
            pltpu.async_copy(item_h.at[iidx.at[pl.ds(j * 128, 128)]],
                             irows.at[pl.ds(j * 128, 128)], sem)

    def drain_gathers(iidx, irows, sem):
        for j in range(_NG):
            pltpu.make_async_copy(
                item_h.at[iidx.at[pl.ds(j * 128, 128)]],
                irows.at[pl.ds(j * 128, 128)], sem).wait()

    def out_dst(chunk):
        return scoresf_o.at[pl.ds((base + chunk * ICH) * NI, _NPAD)]

    def compute(chunk, iidx, irows, sbuf):
        def rowbody(r, _):
            rbase = chunk * ICH + r
            ub = [plsc.load_gather(u_v, [_splat(rbase * D + d)])
                  for d in range(D)]
            na2a = ub[0] * ub[0]
            na2b = ub[1] * ub[1]
            for d in range(2, D, 2):
                na2a = na2a + ub[d] * ub[d]
                na2b = na2b + ub[d + 1] * ub[d + 1]
            na2 = na2a + na2b
            lgs = []
            for gi, goff in enumerate(_GOFFS):
                off = pl.multiple_of(r * NI, 8) + goff
                tbuf = tbufs[gi % 4]
                for k in range(16):
                    rv = irows[off + k, :]
                    plsc.store_scatter(tbuf, [lanes * 16 + k], rv)
                c0 = tbuf[pl.ds(0, 16)]
                c1 = tbuf[pl.ds(16, 16)]
                dot0 = c0 * ub[0]
                dot1 = c1 * ub[1]
                nb0 = c0 * c0
                nb1 = c1 * c1
                for d in range(2, D, 2):
                    c0 = tbuf[pl.ds(d * 16, 16)]
                    c1 = tbuf[pl.ds((d + 1) * 16, 16)]
                    dot0 = dot0 + c0 * ub[d]
                    dot1 = dot1 + c1 * ub[d + 1]
                    nb0 = nb0 + c0 * c0
                    nb1 = nb1 + c1 * c1
                dotacc = dot0 + dot1
                nbacc = nb0 + nb1
                rs = _rsqrt(jnp.maximum(na2 * nbacc, 1e-16))
                lgs.append(dotacc * rs)
            m = lgs[0]
            for lg in lgs[1:]:
                m = jnp.maximum(m, lg)
            mv = jnp.full((16,), jnp.max(m), _f32)
            es = [jnp.exp(lg - mv) for lg in lgs]
            sv = es[0]
            for e in es[1:-1]:
                sv = sv + e
            sv = sv + jnp.where(lanes < 8, 0.0, es[-1])
            rcp = jnp.full((16,), 1.0, _f32) / jnp.full((16,), jnp.sum(sv), _f32)
            for i, goff in enumerate(_GOFFS):
                sbuf[pl.ds(pl.multiple_of(r * NI, 8) + goff, 16)] = es[i] * rcp
            return 0

        lax.fori_loop(0, ICH, rowbody, 0)

    pltpu.async_copy(idx_src(0), iidx_a, semi_a)
    pltpu.async_copy(idx_src(1), iidx_b, semi_b)
    pltpu.make_async_copy(idx_src(0), iidx_a, semi_a).wait()
    fire_gathers(iidx_a, irows_a, semg_a)

    def pipe(cc2, _):
        e = cc2 * 2

        pltpu.make_async_copy(idx_src(1), iidx_b, semi_b).wait()
        fire_gathers(iidx_b, irows_b, semg_b)
        drain_gathers(iidx_a, irows_a, semg_a)

        @pl.when(cc2 < NIC // 2 - 1)
        def _():
            pltpu.async_copy(idx_src(e + 2), iidx_a, semi_a)

        @pl.when(cc2 >= 1)
        def _():
            pltpu.make_async_copy(sbuf_a, out_dst(0), semo_a).wait()
        compute(e, iidx_a, irows_a, sbuf_a)
        pltpu.async_copy(sbuf_a, out_dst(e), semo_a)

        @pl.when(cc2 < NIC // 2 - 1)
        def _():
            pltpu.make_async_copy(idx_src(e + 2), iidx_a, semi_a).wait()
            fire_gathers(iidx_a, irows_a, semg_a)
        drain_gathers(iidx_b, irows_b, semg_b)

        @pl.when(cc2 < NIC // 2 - 1)
        def _():
            pltpu.async_copy(idx_src(e + 3), iidx_b, semi_b)

        @pl.when(cc2 >= 1)
        def _():
            pltpu.make_async_copy(sbuf_b, out_dst(0), semo_b).wait()
        compute(e + 1, iidx_b, irows_b, sbuf_b)
        pltpu.async_copy(sbuf_b, out_dst(e + 1), semo_b)
        return 0

    lax.fori_loop(0, NIC // 2, pipe, 0)
    pltpu.make_async_copy(sbuf_a, out_dst(0), semo_a).wait()
    pltpu.make_async_copy(sbuf_b, out_dst(0), semo_b).wait()


@functools.cache
def _sc_items():
  return pl.kernel(
    _sc_items_body,
    out_type=jax.ShapeDtypeStruct((B * NI,), _f32),
    mesh=plsc.VectorSubcoreMesh(core_axis_name="c", subcore_axis_name="s",
                                num_cores=NC, num_subcores=NS),
    compiler_params=pltpu.CompilerParams(needs_layout_passes=False, use_tc_tiling_on_sc=False),
    scratch_types=[
        pltpu.VMEM((RPW * D,), _f32),
        pltpu.VMEM((_NPAD,), _i32),
        pltpu.VMEM((_NPAD,), _i32),
        pltpu.VMEM((_NPAD, D), _f32),
        pltpu.VMEM((_NPAD, D), _f32),
        pltpu.VMEM((256,), _f32),
        pltpu.VMEM((256,), _f32),
        pltpu.VMEM((256,), _f32),
        pltpu.VMEM((256,), _f32),
        pltpu.VMEM((_NPAD,), _f32),
        pltpu.VMEM((_NPAD,), _f32),
        pltpu.SemaphoreType.DMA,
        pltpu.SemaphoreType.DMA,
        pltpu.SemaphoreType.DMA,
        pltpu.SemaphoreType.DMA,
        pltpu.SemaphoreType.DMA,
        pltpu.SemaphoreType.DMA,
    ],
  )


_MBLK = 2048


def _mlp_body(user_r, prov_r, city_r, hist_r, cont_r,
              w1_r, b1_r, w2_r, b2_r, w3_r, b3_r, out_r):
    w1 = w1_r[...]
    h = (jnp.dot(user_r[...], w1[0:16, :], preferred_element_type=_f32)
         + jnp.dot(prov_r[...], w1[16:32, :], preferred_element_type=_f32)
         + jnp.dot(city_r[...], w1[32:48, :], preferred_element_type=_f32)
         + jnp.dot(hist_r[...], w1[48:64, :], preferred_element_type=_f32)
         + jnp.dot(cont_r[...], w1[64:92, :], preferred_element_type=_f32)
         + b1_r[...])
    h = jnp.dot(h, w2_r[...], preferred_element_type=_f32) + b2_r[...]
    out_r[...] = jnp.dot(h, w3_r[...], preferred_element_type=_f32) + b3_r[...]


def _row_spec(cols):
    return pl.BlockSpec((_MBLK, cols), lambda i: (i, 0))


def _full_spec(shape):
    return pl.BlockSpec(shape, lambda i: tuple(0 for _ in shape))


_tc_mlp = pl.pallas_call(
    _mlp_body,
    grid=(B // _MBLK,),
    in_specs=[
        _row_spec(D), _row_spec(D), _row_spec(D), _row_spec(D), _row_spec(28),
        _full_spec((92, 128)), _full_spec((1, 128)),
        _full_spec((128, 32)), _full_spec((1, 32)),
        _full_spec((32, 16)), _full_spec((1, 16)),
    ],
    out_specs=_row_spec(D),
    out_shape=jax.ShapeDtypeStruct((B, D), _f32),
)


def kernel(disc, cont, history, items, item_table, user_table, city_table,
           prov_table, dev_table, os_table, W1, b1, W2, b2, W3, b3):
    it0 = item_table.at[0].set(0.0)
    user_e, prov_e, city_e, histf_e = _sc_gather()(
        disc.reshape(-1), history.reshape(-1), it0, user_table,
        prov_table, city_table)
    u16 = _tc_mlp(user_e, prov_e, city_e, histf_e.reshape(B, D), cont,
                  W1, b1.reshape(1, -1), W2, b2.reshape(1, -1),
                  W3, b3.reshape(1, -1))
    scoresf = _sc_items()(items.reshape(-1), it0, u16.reshape(-1))
    return scoresf.reshape(B, NI)
